# baseline (speedup 1.0000x reference)
import functools

import jax
import jax.numpy as jnp
from jax import lax
from jax.experimental import pallas as pl
from jax.experimental.pallas import tpu as pltpu
from jax.experimental.pallas import tpu_sc as plsc

LANES = 16
NC = 2
NS = 16
NW = NC * NS
SUB = 8
WIN = 64
CHUNK = 128


def _bitrev4(x):
    return ((x & 1) << 3) | ((x & 2) << 1) | ((x & 4) >> 1) | ((x & 8) >> 3)


def _shuffle(a, idx):
    return lax.gather(
        a, idx[:, None],
        lax.GatherDimensionNumbers(
            offset_dims=(), collapsed_slice_dims=(0,), start_index_map=(0,)),
        slice_sizes=(1,),
        mode=lax.GatherScatterMode.PROMISE_IN_BOUNDS)


@functools.lru_cache(maxsize=None)
def _make_compact_kernel(V, D):
    TILES = V // SUB
    ROWS_W = WIN * SUB
    OUT_W = ROWS_W * D // 128

    mesh = plsc.VectorSubcoreMesh(core_axis_name="c", subcore_axis_name="s")
    out_sds = jax.ShapeDtypeStruct((V * D // 128, 128), jnp.float32)

    @functools.partial(
        pl.kernel,
        mesh=mesh,
        out_type=(out_sds, out_sds),
        compiler_params=pltpu.CompilerParams(
            needs_layout_passes=False, use_tc_tiling_on_sc=True),
        scratch_types=[
            pltpu.VMEM((ROWS_W, D), jnp.float32),
            pltpu.VMEM((OUT_W, 128), jnp.float32),
            pltpu.SemaphoreType.DMA,
        ],
    )
    def compact_kernel(uf_hbm, mf_hbm, cu_hbm, cm_hbm,
                       stage_v, packed_v, sem):
        wid = lax.axis_index("s") * NC + lax.axis_index("c")
        t0 = (wid * TILES) // NW // 4 * 4
        t1 = jnp.where(wid == NW - 1, TILES, ((wid + 1) * TILES) // NW // 4 * 4)
        n_win = (t1 - t0 + WIN - 1) // WIN

        def do_table(src_hbm, dst_hbm):
            def window_body(i, carry):
                t = jnp.minimum(t0 + i * WIN, t1 - WIN)
                r0 = t * SUB
                pltpu.async_copy(
                    src_hbm.at[pl.ds(r0, ROWS_W), :], stage_v, sem).wait()

                def pack_body(q, c2):
                    for j in range(4):
                        r = q * 4 + j
                        packed_v[q, pl.ds(j * D, LANES)] = (
                            stage_v[r, pl.ds(0, LANES)])
                        packed_v[q, pl.ds(j * D + LANES, LANES)] = (
                            stage_v[r, pl.ds(LANES, LANES)])
                    return c2

                lax.fori_loop(0, OUT_W, pack_body, 0)
                pltpu.async_copy(
                    packed_v, dst_hbm.at[pl.ds(t * 2, OUT_W), :], sem).wait()
                return carry

            lax.fori_loop(0, n_win, window_body, 0)

        do_table(uf_hbm, cu_hbm)
        do_table(mf_hbm, cm_hbm)

    return compact_kernel


@functools.lru_cache(maxsize=None)
def _make_gather_kernel(B, V, D):
    bw = B // NW
    n_groups = bw // LANES
    n_chunks = bw // CHUNK
    groups_per_chunk = CHUNK // LANES

    mesh = plsc.VectorSubcoreMesh(core_axis_name="c", subcore_axis_name="s")

    @functools.partial(
        pl.kernel,
        mesh=mesh,
        out_type=jax.ShapeDtypeStruct((B,), jnp.float32),
        compiler_params=pltpu.CompilerParams(
            needs_layout_passes=False, use_tc_tiling_on_sc=True),
        scratch_types=[
            pltpu.VMEM((2 * bw,), jnp.int32),
            pltpu.VMEM((n_chunks, CHUNK), jnp.int32),
            pltpu.VMEM((n_chunks, CHUNK), jnp.int32),
            pltpu.VMEM((bw,), jnp.int32),
            pltpu.VMEM((bw,), jnp.int32),
            pltpu.VMEM((CHUNK, 128), jnp.float32),
            pltpu.VMEM((CHUNK, 128), jnp.float32),
            pltpu.VMEM((bw,), jnp.float32),
            pltpu.SemaphoreType.DMA,
        ],
    )
    def gather_kernel(data_hbm, cu_hbm, cm_hbm, out_hbm,
                      data_v, ug_v, mg_v, uo_v, mo_v,
                      urows_v, mrows_v, out_v, sem):
        wid = lax.axis_index("s") * NC + lax.axis_index("c")
        base = wid * bw

        pltpu.sync_copy(data_hbm.at[pl.ds(2 * base, 2 * bw)], data_v)

        iota = lax.iota(jnp.int32, LANES)
        iota2 = iota * 2
        three = jnp.full((LANES,), 3, jnp.int32)
        for j in range(n_groups):
            u = plsc.load_gather(data_v, [iota2 + (2 * j * LANES)])
            m = plsc.load_gather(data_v, [iota2 + (2 * j * LANES + 1)])
            c, o = divmod(j * LANES, CHUNK)
            ug_v[c, pl.ds(o, LANES)] = lax.shift_right_logical(u, 2)
            mg_v[c, pl.ds(o, LANES)] = lax.shift_right_logical(m, 2)
            dst = pl.ds(j * LANES, LANES)
            uo_v[dst] = lax.bitwise_and(u, three) * D
            mo_v[dst] = lax.bitwise_and(m, three) * D

        for c in range(n_chunks):
            cp_u = pltpu.async_copy(cu_hbm.at[ug_v.at[c]], urows_v, sem)
            cp_m = pltpu.async_copy(cm_hbm.at[mg_v.at[c]], mrows_v, sem)
            cp_u.wait()
            cp_m.wait()
            for g in range(groups_per_chunk):
                off = c * CHUNK + g * LANES
                uo = uo_v[pl.ds(off, LANES)]
                mo = mo_v[pl.ds(off, LANES)]
                prods = [None] * LANES
                for t in range(LANES):
                    slot = g * LANES + t
                    p0 = (urows_v[slot, pl.ds(uo[t], LANES)]
                          * mrows_v[slot, pl.ds(mo[t], LANES)])
                    p1 = (urows_v[slot, pl.ds(uo[t] + LANES, LANES)]
                          * mrows_v[slot, pl.ds(mo[t] + LANES, LANES)])
                    prods[_bitrev4(t)] = p0 + p1
                k = LANES
                while len(prods) > 1:
                    k //= 2
                    kvec = jnp.full((LANES,), k, jnp.int32)
                    idx = lax.bitwise_xor(iota, kvec)
                    sel = lax.bitwise_and(iota, kvec) == jnp.zeros(
                        (LANES,), jnp.int32)
                    merged = []
                    for i in range(len(prods) // 2):
                        a = prods[2 * i]
                        b = prods[2 * i + 1]
                        a2 = a + _shuffle(a, idx)
                        b2 = b + _shuffle(b, idx)
                        merged.append(jnp.where(sel, a2, b2))
                    prods = merged
                out_v[pl.ds(off, LANES)] = prods[0]

        pltpu.sync_copy(out_v, out_hbm.at[pl.ds(base, bw)])

    return gather_kernel


def kernel(data, user_factors, movie_factors):
    data_flat = data.astype(jnp.int32).reshape(-1)
    B = data.shape[0]
    V, D = user_factors.shape
    cu, cm = _make_compact_kernel(V, D)(user_factors, movie_factors)
    return _make_gather_kernel(B, V, D)(data_flat, cu, cm)


# speedup vs baseline: 1.3089x; 1.3089x over previous
import functools

import jax
import jax.numpy as jnp
from jax import lax
from jax.experimental import pallas as pl
from jax.experimental.pallas import tpu as pltpu
from jax.experimental.pallas import tpu_sc as plsc

LANES = 16
NC = 2
NS = 16
NW = NC * NS
SUB = 8
WIN = 64
CHUNK = 128


def _bitrev4(x):
    return ((x & 1) << 3) | ((x & 2) << 1) | ((x & 4) >> 1) | ((x & 8) >> 3)


def _shuffle(a, idx):
    return lax.gather(
        a, idx[:, None],
        lax.GatherDimensionNumbers(
            offset_dims=(), collapsed_slice_dims=(0,), start_index_map=(0,)),
        slice_sizes=(1,),
        mode=lax.GatherScatterMode.PROMISE_IN_BOUNDS)


@functools.lru_cache(maxsize=None)
def _make_compact_kernel(V, D):
    TILES = V // SUB
    ROWS_W = WIN * SUB
    OUT_W = ROWS_W * D // 128

    mesh = plsc.VectorSubcoreMesh(core_axis_name="c", subcore_axis_name="s")
    out_sds = jax.ShapeDtypeStruct((V * D // 128, 128), jnp.float32)

    @functools.partial(
        pl.kernel,
        mesh=mesh,
        out_type=(out_sds, out_sds),
        compiler_params=pltpu.CompilerParams(
            needs_layout_passes=False, use_tc_tiling_on_sc=True),
        scratch_types=[
            pltpu.VMEM((ROWS_W, D), jnp.float32),
            pltpu.VMEM((OUT_W, 128), jnp.float32),
            pltpu.SemaphoreType.DMA,
        ],
    )
    def compact_kernel(uf_hbm, mf_hbm, cu_hbm, cm_hbm,
                       stage_v, packed_v, sem):
        wid = lax.axis_index("s") * NC + lax.axis_index("c")
        t0 = (wid * TILES) // NW // 4 * 4
        t1 = jnp.where(wid == NW - 1, TILES, ((wid + 1) * TILES) // NW // 4 * 4)
        n_win = (t1 - t0 + WIN - 1) // WIN

        def do_table(src_hbm, dst_hbm):
            def window_body(i, carry):
                t = jnp.minimum(t0 + i * WIN, t1 - WIN)
                r0 = t * SUB
                pltpu.async_copy(
                    src_hbm.at[pl.ds(r0, ROWS_W), :], stage_v, sem).wait()

                for q in range(OUT_W):
                    for j in range(4):
                        r = q * 4 + j
                        packed_v[q, pl.ds(j * D, LANES)] = (
                            stage_v[r, pl.ds(0, LANES)])
                        packed_v[q, pl.ds(j * D + LANES, LANES)] = (
                            stage_v[r, pl.ds(LANES, LANES)])
                pltpu.async_copy(
                    packed_v, dst_hbm.at[pl.ds(t * 2, OUT_W), :], sem).wait()
                return carry

            lax.fori_loop(0, n_win, window_body, 0)

        do_table(uf_hbm, cu_hbm)
        do_table(mf_hbm, cm_hbm)

    return compact_kernel


@functools.lru_cache(maxsize=None)
def _make_gather_kernel(B, V, D):
    bw = B // NW
    n_groups = bw // LANES
    n_chunks = bw // CHUNK
    groups_per_chunk = CHUNK // LANES

    mesh = plsc.VectorSubcoreMesh(core_axis_name="c", subcore_axis_name="s")

    @functools.partial(
        pl.kernel,
        mesh=mesh,
        out_type=jax.ShapeDtypeStruct((B,), jnp.float32),
        compiler_params=pltpu.CompilerParams(
            needs_layout_passes=False, use_tc_tiling_on_sc=True),
        scratch_types=[
            pltpu.VMEM((2 * bw,), jnp.int32),
            pltpu.VMEM((n_chunks, CHUNK), jnp.int32),
            pltpu.VMEM((n_chunks, CHUNK), jnp.int32),
            pltpu.VMEM((bw,), jnp.int32),
            pltpu.VMEM((bw,), jnp.int32),
            pltpu.VMEM((CHUNK, 128), jnp.float32),
            pltpu.VMEM((CHUNK, 128), jnp.float32),
            pltpu.VMEM((bw,), jnp.float32),
            pltpu.SemaphoreType.DMA,
        ],
    )
    def gather_kernel(data_hbm, cu_hbm, cm_hbm, out_hbm,
                      data_v, ug_v, mg_v, uo_v, mo_v,
                      urows_v, mrows_v, out_v, sem):
        wid = lax.axis_index("s") * NC + lax.axis_index("c")
        base = wid * bw

        pltpu.sync_copy(data_hbm.at[pl.ds(2 * base, 2 * bw)], data_v)

        iota = lax.iota(jnp.int32, LANES)
        iota2 = iota * 2
        three = jnp.full((LANES,), 3, jnp.int32)
        for j in range(n_groups):
            u = plsc.load_gather(data_v, [iota2 + (2 * j * LANES)])
            m = plsc.load_gather(data_v, [iota2 + (2 * j * LANES + 1)])
            c, o = divmod(j * LANES, CHUNK)
            ug_v[c, pl.ds(o, LANES)] = lax.shift_right_logical(u, 2)
            mg_v[c, pl.ds(o, LANES)] = lax.shift_right_logical(m, 2)
            dst = pl.ds(j * LANES, LANES)
            uo_v[dst] = lax.bitwise_and(u, three) * D
            mo_v[dst] = lax.bitwise_and(m, three) * D

        for c in range(n_chunks):
            cp_u = pltpu.async_copy(cu_hbm.at[ug_v.at[c]], urows_v, sem)
            cp_m = pltpu.async_copy(cm_hbm.at[mg_v.at[c]], mrows_v, sem)
            cp_u.wait()
            cp_m.wait()
            for g in range(groups_per_chunk):
                off = c * CHUNK + g * LANES
                uo = uo_v[pl.ds(off, LANES)]
                mo = mo_v[pl.ds(off, LANES)]
                prods = [None] * LANES
                for t in range(LANES):
                    slot = g * LANES + t
                    p0 = (urows_v[slot, pl.ds(uo[t], LANES)]
                          * mrows_v[slot, pl.ds(mo[t], LANES)])
                    p1 = (urows_v[slot, pl.ds(uo[t] + LANES, LANES)]
                          * mrows_v[slot, pl.ds(mo[t] + LANES, LANES)])
                    prods[_bitrev4(t)] = p0 + p1
                k = LANES
                while len(prods) > 1:
                    k //= 2
                    kvec = jnp.full((LANES,), k, jnp.int32)
                    idx = lax.bitwise_xor(iota, kvec)
                    sel = lax.bitwise_and(iota, kvec) == jnp.zeros(
                        (LANES,), jnp.int32)
                    merged = []
                    for i in range(len(prods) // 2):
                        a = prods[2 * i]
                        b = prods[2 * i + 1]
                        a2 = a + _shuffle(a, idx)
                        b2 = b + _shuffle(b, idx)
                        merged.append(jnp.where(sel, a2, b2))
                    prods = merged
                out_v[pl.ds(off, LANES)] = prods[0]

        pltpu.sync_copy(out_v, out_hbm.at[pl.ds(base, bw)])

    return gather_kernel


def kernel(data, user_factors, movie_factors):
    data_flat = data.astype(jnp.int32).reshape(-1)
    B = data.shape[0]
    V, D = user_factors.shape
    cu, cm = _make_compact_kernel(V, D)(user_factors, movie_factors)
    return _make_gather_kernel(B, V, D)(data_flat, cu, cm)


# speedup vs baseline: 1.6653x; 1.2723x over previous
import functools

import jax
import jax.numpy as jnp
from jax import lax
from jax.experimental import pallas as pl
from jax.experimental.pallas import tpu as pltpu
from jax.experimental.pallas import tpu_sc as plsc

LANES = 16
NC = 2
NS = 16
NW = NC * NS
SUB = 8
WIN = 64
CHUNK = 128


def _bitrev4(x):
    return ((x & 1) << 3) | ((x & 2) << 1) | ((x & 4) >> 1) | ((x & 8) >> 3)


def _shuffle(a, idx):
    return lax.gather(
        a, idx[:, None],
        lax.GatherDimensionNumbers(
            offset_dims=(), collapsed_slice_dims=(0,), start_index_map=(0,)),
        slice_sizes=(1,),
        mode=lax.GatherScatterMode.PROMISE_IN_BOUNDS)


@functools.lru_cache(maxsize=None)
def _make_compact_kernel(V, D):
    TILES = V // SUB
    ROWS_W = WIN * SUB
    OUT_W = ROWS_W * D // 128

    mesh = plsc.VectorSubcoreMesh(core_axis_name="c", subcore_axis_name="s")
    out_sds = jax.ShapeDtypeStruct((V * D // 128, 128), jnp.float32)

    @functools.partial(
        pl.kernel,
        mesh=mesh,
        out_type=(out_sds, out_sds),
        compiler_params=pltpu.CompilerParams(
            needs_layout_passes=False, use_tc_tiling_on_sc=True),
        scratch_types=[
            pltpu.VMEM((WIN, SUB, D), jnp.float32),
            pltpu.VMEM((OUT_W, 128), jnp.float32),
            pltpu.SemaphoreType.DMA,
        ],
    )
    def compact_kernel(uf_hbm, mf_hbm, cu_hbm, cm_hbm,
                       stage_v, packed_v, sem):
        wid = lax.axis_index("s") * NC + lax.axis_index("c")
        t0 = (wid * TILES) // NW // 4 * 4
        t1 = jnp.where(wid == NW - 1, TILES, ((wid + 1) * TILES) // NW // 4 * 4)
        n_win = (t1 - t0 + WIN - 1) // WIN

        def do_table(src_hbm, dst_hbm):
            def window_body(i, carry):
                t = jnp.minimum(t0 + i * WIN, t1 - WIN)
                pltpu.async_copy(
                    src_hbm.at[pl.ds(t, WIN)], stage_v, sem).wait()

                for q in range(OUT_W):
                    for j in range(4):
                        r = q * 4 + j
                        tt, s = divmod(r, SUB)
                        packed_v[q, pl.ds(j * D, LANES)] = (
                            stage_v[tt, s, pl.ds(0, LANES)])
                        packed_v[q, pl.ds(j * D + LANES, LANES)] = (
                            stage_v[tt, s, pl.ds(LANES, LANES)])
                pltpu.async_copy(
                    packed_v, dst_hbm.at[pl.ds(t * 2, OUT_W), :], sem).wait()
                return carry

            lax.fori_loop(0, n_win, window_body, 0)

        do_table(uf_hbm, cu_hbm)
        do_table(mf_hbm, cm_hbm)

    return compact_kernel


@functools.lru_cache(maxsize=None)
def _make_gather_kernel(B, V, D):
    bw = B // NW
    n_groups = bw // LANES
    n_chunks = bw // CHUNK
    groups_per_chunk = CHUNK // LANES

    mesh = plsc.VectorSubcoreMesh(core_axis_name="c", subcore_axis_name="s")

    @functools.partial(
        pl.kernel,
        mesh=mesh,
        out_type=jax.ShapeDtypeStruct((B,), jnp.float32),
        compiler_params=pltpu.CompilerParams(
            needs_layout_passes=False, use_tc_tiling_on_sc=True),
        scratch_types=[
            pltpu.VMEM((2 * bw,), jnp.int32),
            pltpu.VMEM((n_chunks, CHUNK), jnp.int32),
            pltpu.VMEM((n_chunks, CHUNK), jnp.int32),
            pltpu.VMEM((bw,), jnp.int32),
            pltpu.VMEM((bw,), jnp.int32),
            pltpu.VMEM((CHUNK, 128), jnp.float32),
            pltpu.VMEM((CHUNK, 128), jnp.float32),
            pltpu.VMEM((bw,), jnp.float32),
            pltpu.SemaphoreType.DMA,
        ],
    )
    def gather_kernel(data_hbm, cu_hbm, cm_hbm, out_hbm,
                      data_v, ug_v, mg_v, uo_v, mo_v,
                      urows_v, mrows_v, out_v, sem):
        wid = lax.axis_index("s") * NC + lax.axis_index("c")
        base = wid * bw

        pltpu.sync_copy(data_hbm.at[pl.ds(2 * base, 2 * bw)], data_v)

        iota = lax.iota(jnp.int32, LANES)
        iota2 = iota * 2
        three = jnp.full((LANES,), 3, jnp.int32)
        for j in range(n_groups):
            u = plsc.load_gather(data_v, [iota2 + (2 * j * LANES)])
            m = plsc.load_gather(data_v, [iota2 + (2 * j * LANES + 1)])
            c, o = divmod(j * LANES, CHUNK)
            ug_v[c, pl.ds(o, LANES)] = lax.shift_right_logical(u, 2)
            mg_v[c, pl.ds(o, LANES)] = lax.shift_right_logical(m, 2)
            dst = pl.ds(j * LANES, LANES)
            uo_v[dst] = lax.bitwise_and(u, three) * D
            mo_v[dst] = lax.bitwise_and(m, three) * D

        for c in range(n_chunks):
            cp_u = pltpu.async_copy(cu_hbm.at[ug_v.at[c]], urows_v, sem)
            cp_m = pltpu.async_copy(cm_hbm.at[mg_v.at[c]], mrows_v, sem)
            cp_u.wait()
            cp_m.wait()
            for g in range(groups_per_chunk):
                off = c * CHUNK + g * LANES
                uo = uo_v[pl.ds(off, LANES)]
                mo = mo_v[pl.ds(off, LANES)]
                prods = [None] * LANES
                for t in range(LANES):
                    slot = g * LANES + t
                    p0 = (urows_v[slot, pl.ds(uo[t], LANES)]
                          * mrows_v[slot, pl.ds(mo[t], LANES)])
                    p1 = (urows_v[slot, pl.ds(uo[t] + LANES, LANES)]
                          * mrows_v[slot, pl.ds(mo[t] + LANES, LANES)])
                    prods[_bitrev4(t)] = p0 + p1
                k = LANES
                while len(prods) > 1:
                    k //= 2
                    kvec = jnp.full((LANES,), k, jnp.int32)
                    idx = lax.bitwise_xor(iota, kvec)
                    sel = lax.bitwise_and(iota, kvec) == jnp.zeros(
                        (LANES,), jnp.int32)
                    merged = []
                    for i in range(len(prods) // 2):
                        a = prods[2 * i]
                        b = prods[2 * i + 1]
                        a2 = a + _shuffle(a, idx)
                        b2 = b + _shuffle(b, idx)
                        merged.append(jnp.where(sel, a2, b2))
                    prods = merged
                out_v[pl.ds(off, LANES)] = prods[0]

        pltpu.sync_copy(out_v, out_hbm.at[pl.ds(base, bw)])

    return gather_kernel


def kernel(data, user_factors, movie_factors):
    data_flat = data.astype(jnp.int32).reshape(-1)
    B = data.shape[0]
    V, D = user_factors.shape
    uf3 = user_factors.reshape(V // SUB, SUB, D)
    mf3 = movie_factors.reshape(V // SUB, SUB, D)
    cu, cm = _make_compact_kernel(V, D)(uf3, mf3)
    return _make_gather_kernel(B, V, D)(data_flat, cu, cm)


# speedup vs baseline: 1.8404x; 1.1052x over previous
import functools

import jax
import jax.numpy as jnp
from jax import lax
from jax.experimental import pallas as pl
from jax.experimental.pallas import tpu as pltpu
from jax.experimental.pallas import tpu_sc as plsc

LANES = 16
NC = 2
NS = 16
NW = NC * NS
SUB = 8
WIN = 32
CHUNK = 128


def _bitrev4(x):
    return ((x & 1) << 3) | ((x & 2) << 1) | ((x & 4) >> 1) | ((x & 8) >> 3)


def _shuffle(a, idx):
    return lax.gather(
        a, idx[:, None],
        lax.GatherDimensionNumbers(
            offset_dims=(), collapsed_slice_dims=(0,), start_index_map=(0,)),
        slice_sizes=(1,),
        mode=lax.GatherScatterMode.PROMISE_IN_BOUNDS)


@functools.lru_cache(maxsize=None)
def _make_compact_kernel(V, D):
    TILES = V // SUB
    ROWS_W = WIN * SUB
    OUT_W = ROWS_W * D // 128

    mesh = plsc.VectorSubcoreMesh(core_axis_name="c", subcore_axis_name="s")
    out_sds = jax.ShapeDtypeStruct((V * D // 128, 128), jnp.float32)

    @functools.partial(
        pl.kernel,
        mesh=mesh,
        out_type=(out_sds, out_sds),
        compiler_params=pltpu.CompilerParams(
            needs_layout_passes=False, use_tc_tiling_on_sc=True),
        scratch_types=[
            pltpu.VMEM((WIN, SUB, D), jnp.float32),
            pltpu.VMEM((WIN, SUB, D), jnp.float32),
            pltpu.VMEM((OUT_W, 128), jnp.float32),
            pltpu.VMEM((OUT_W, 128), jnp.float32),
            pltpu.SemaphoreType.DMA,
            pltpu.SemaphoreType.DMA,
            pltpu.SemaphoreType.DMA,
        ],
    )
    def compact_kernel(uf_hbm, mf_hbm, cu_hbm, cm_hbm,
                       s0_v, s1_v, p0_v, p1_v, sem_a, sem_b, sem_o):
        wid = lax.axis_index("s") * NC + lax.axis_index("c")
        t0 = (wid * TILES) // NW // 4 * 4
        t1 = jnp.where(wid == NW - 1, TILES, ((wid + 1) * TILES) // NW // 4 * 4)
        n_win = (t1 - t0 + WIN - 1) // WIN
        n_pairs = (n_win + 1) // 2

        def repack(stage_v, packed_v):
            for q in range(OUT_W):
                for j in range(4):
                    r = q * 4 + j
                    tt, s = divmod(r, SUB)
                    packed_v[q, pl.ds(j * D, LANES)] = (
                        stage_v[tt, s, pl.ds(0, LANES)])
                    packed_v[q, pl.ds(j * D + LANES, LANES)] = (
                        stage_v[tt, s, pl.ds(LANES, LANES)])

        def do_table(src_hbm, dst_hbm):
            def t_at(k):
                return jnp.minimum(t0 + k * WIN, t1 - WIN)

            def wait_stage(stage_v, sem):
                pltpu.make_async_copy(
                    src_hbm.at[pl.ds(t0, WIN)], stage_v, sem).wait()

            pltpu.async_copy(src_hbm.at[pl.ds(t_at(0), WIN)], s0_v, sem_a)

            def pair_body(i, carry):
                te, to = t_at(2 * i), t_at(2 * i + 1)
                pltpu.async_copy(src_hbm.at[pl.ds(to, WIN)], s1_v, sem_b)
                wait_stage(s0_v, sem_a)
                repack(s0_v, p0_v)
                out0 = pltpu.async_copy(
                    p0_v, dst_hbm.at[pl.ds(te * 2, OUT_W), :], sem_o)
                pltpu.async_copy(
                    src_hbm.at[pl.ds(t_at(2 * i + 2), WIN)], s0_v, sem_a)
                wait_stage(s1_v, sem_b)
                repack(s1_v, p1_v)
                out1 = pltpu.async_copy(
                    p1_v, dst_hbm.at[pl.ds(to * 2, OUT_W), :], sem_o)
                out0.wait()
                out1.wait()
                return carry

            lax.fori_loop(0, n_pairs, pair_body, 0)
            wait_stage(s0_v, sem_a)

        do_table(uf_hbm, cu_hbm)
        do_table(mf_hbm, cm_hbm)

    return compact_kernel


@functools.lru_cache(maxsize=None)
def _make_gather_kernel(B, V, D):
    bw = B // NW
    n_groups = bw // LANES
    n_chunks = bw // CHUNK
    groups_per_chunk = CHUNK // LANES

    mesh = plsc.VectorSubcoreMesh(core_axis_name="c", subcore_axis_name="s")

    @functools.partial(
        pl.kernel,
        mesh=mesh,
        out_type=jax.ShapeDtypeStruct((B,), jnp.float32),
        compiler_params=pltpu.CompilerParams(
            needs_layout_passes=False, use_tc_tiling_on_sc=True),
        scratch_types=[
            pltpu.VMEM((2 * bw,), jnp.int32),
            pltpu.VMEM((n_chunks, CHUNK), jnp.int32),
            pltpu.VMEM((n_chunks, CHUNK), jnp.int32),
            pltpu.VMEM((bw,), jnp.int32),
            pltpu.VMEM((bw,), jnp.int32),
            pltpu.VMEM((CHUNK, 128), jnp.float32),
            pltpu.VMEM((CHUNK, 128), jnp.float32),
            pltpu.VMEM((bw,), jnp.float32),
            pltpu.SemaphoreType.DMA,
        ],
    )
    def gather_kernel(data_hbm, cu_hbm, cm_hbm, out_hbm,
                      data_v, ug_v, mg_v, uo_v, mo_v,
                      urows_v, mrows_v, out_v, sem):
        wid = lax.axis_index("s") * NC + lax.axis_index("c")
        base = wid * bw

        pltpu.sync_copy(data_hbm.at[pl.ds(2 * base, 2 * bw)], data_v)

        iota = lax.iota(jnp.int32, LANES)
        iota2 = iota * 2
        three = jnp.full((LANES,), 3, jnp.int32)
        for j in range(n_groups):
            u = plsc.load_gather(data_v, [iota2 + (2 * j * LANES)])
            m = plsc.load_gather(data_v, [iota2 + (2 * j * LANES + 1)])
            c, o = divmod(j * LANES, CHUNK)
            ug_v[c, pl.ds(o, LANES)] = lax.shift_right_logical(u, 2)
            mg_v[c, pl.ds(o, LANES)] = lax.shift_right_logical(m, 2)
            dst = pl.ds(j * LANES, LANES)
            uo_v[dst] = lax.bitwise_and(u, three) * D
            mo_v[dst] = lax.bitwise_and(m, three) * D

        for c in range(n_chunks):
            cp_u = pltpu.async_copy(cu_hbm.at[ug_v.at[c]], urows_v, sem)
            cp_m = pltpu.async_copy(cm_hbm.at[mg_v.at[c]], mrows_v, sem)
            cp_u.wait()
            cp_m.wait()
            for g in range(groups_per_chunk):
                off = c * CHUNK + g * LANES
                uo = uo_v[pl.ds(off, LANES)]
                mo = mo_v[pl.ds(off, LANES)]
                prods = [None] * LANES
                for t in range(LANES):
                    slot = g * LANES + t
                    p0 = (urows_v[slot, pl.ds(uo[t], LANES)]
                          * mrows_v[slot, pl.ds(mo[t], LANES)])
                    p1 = (urows_v[slot, pl.ds(uo[t] + LANES, LANES)]
                          * mrows_v[slot, pl.ds(mo[t] + LANES, LANES)])
                    prods[_bitrev4(t)] = p0 + p1
                k = LANES
                while len(prods) > 1:
                    k //= 2
                    kvec = jnp.full((LANES,), k, jnp.int32)
                    idx = lax.bitwise_xor(iota, kvec)
                    sel = lax.bitwise_and(iota, kvec) == jnp.zeros(
                        (LANES,), jnp.int32)
                    merged = []
                    for i in range(len(prods) // 2):
                        a = prods[2 * i]
                        b = prods[2 * i + 1]
                        a2 = a + _shuffle(a, idx)
                        b2 = b + _shuffle(b, idx)
                        merged.append(jnp.where(sel, a2, b2))
                    prods = merged
                out_v[pl.ds(off, LANES)] = prods[0]

        pltpu.sync_copy(out_v, out_hbm.at[pl.ds(base, bw)])

    return gather_kernel


def kernel(data, user_factors, movie_factors):
    data_flat = data.astype(jnp.int32).reshape(-1)
    B = data.shape[0]
    V, D = user_factors.shape
    uf3 = user_factors.reshape(V // SUB, SUB, D)
    mf3 = movie_factors.reshape(V // SUB, SUB, D)
    cu, cm = _make_compact_kernel(V, D)(uf3, mf3)
    return _make_gather_kernel(B, V, D)(data_flat, cu, cm)


# speedup vs baseline: 2.5058x; 1.3616x over previous
import functools

import jax
import jax.numpy as jnp
from jax import lax
from jax.experimental import pallas as pl
from jax.experimental.pallas import tpu as pltpu
from jax.experimental.pallas import tpu_sc as plsc

LANES = 16
NC = 2
NS = 16
CHUNK = 32


def _bitrev4(x):
    return ((x & 1) << 3) | ((x & 2) << 1) | ((x & 4) >> 1) | ((x & 8) >> 3)


def _shuffle(a, idx):
    return lax.gather(
        a, idx[:, None],
        lax.GatherDimensionNumbers(
            offset_dims=(), collapsed_slice_dims=(0,), start_index_map=(0,)),
        slice_sizes=(1,),
        mode=lax.GatherScatterMode.PROMISE_IN_BOUNDS)


@functools.lru_cache(maxsize=None)
def _make_sc_kernel(B, V, D):
    NW = NC * NS
    bw = B // NW
    n_groups = bw // LANES
    n_chunks = bw // CHUNK
    groups_per_chunk = CHUNK // LANES

    mesh = plsc.VectorSubcoreMesh(core_axis_name="c", subcore_axis_name="s")

    @functools.partial(
        pl.kernel,
        mesh=mesh,
        out_type=jax.ShapeDtypeStruct((B,), jnp.float32),
        compiler_params=pltpu.CompilerParams(
            needs_layout_passes=False, use_tc_tiling_on_sc=True),
        scratch_types=[
            pltpu.VMEM((2 * bw,), jnp.int32),
            pltpu.VMEM((bw,), jnp.int32),
            pltpu.VMEM((bw,), jnp.int32),
            pltpu.VMEM((CHUNK, 32), jnp.float32),
            pltpu.VMEM((CHUNK, 32), jnp.float32),
            pltpu.VMEM((bw,), jnp.float32),
            pltpu.SemaphoreType.DMA,
        ],
    )
    def sc_kernel(data_hbm, uf_hbm, mf_hbm, out_hbm,
                  data_v, uidx_v, midx_v, urows_v, mrows_v, out_v, sem):
        wid = lax.axis_index("s") * NC + lax.axis_index("c")
        base = wid * bw

        pltpu.sync_copy(data_hbm.at[pl.ds(2 * base, 2 * bw)], data_v)

        iota = lax.iota(jnp.int32, LANES)
        iota2 = iota * 2
        for j in range(n_groups):
            u = plsc.load_gather(data_v, [iota2 + (2 * j * LANES)])
            m = plsc.load_gather(data_v, [iota2 + (2 * j * LANES + 1)])
            dst = pl.ds(j * LANES, LANES)
            uidx_v[dst] = u
            midx_v[dst] = m

        def chunk_body(c, carry):
            off = c * CHUNK
            copies = []
            for g in range(groups_per_chunk):
                uvec = uidx_v[pl.ds(off + g * LANES, LANES)]
                mvec = midx_v[pl.ds(off + g * LANES, LANES)]
                for t in range(LANES):
                    slot = g * LANES + t
                    copies.append(pltpu.async_copy(
                        uf_hbm.at[uvec[t]], urows_v.at[slot], sem))
                    copies.append(pltpu.async_copy(
                        mf_hbm.at[mvec[t]], mrows_v.at[slot], sem))
            for cp in copies:
                cp.wait()
            for g in range(groups_per_chunk):
                prods = [None] * LANES
                for t in range(LANES):
                    slot = g * LANES + t
                    p0 = (urows_v[slot, pl.ds(0, LANES)]
                          * mrows_v[slot, pl.ds(0, LANES)])
                    p1 = (urows_v[slot, pl.ds(LANES, LANES)]
                          * mrows_v[slot, pl.ds(LANES, LANES)])
                    prods[_bitrev4(t)] = p0 + p1
                k = LANES
                while len(prods) > 1:
                    k //= 2
                    kvec = jnp.full((LANES,), k, jnp.int32)
                    idx = lax.bitwise_xor(iota, kvec)
                    sel = lax.bitwise_and(iota, kvec) == jnp.zeros(
                        (LANES,), jnp.int32)
                    merged = []
                    for i in range(len(prods) // 2):
                        a = prods[2 * i]
                        b = prods[2 * i + 1]
                        a2 = a + _shuffle(a, idx)
                        b2 = b + _shuffle(b, idx)
                        merged.append(jnp.where(sel, a2, b2))
                    prods = merged
                out_v[pl.ds(off + g * LANES, LANES)] = prods[0]
            return carry

        lax.fori_loop(0, n_chunks, chunk_body, 0)
        pltpu.sync_copy(out_v, out_hbm.at[pl.ds(base, bw)])

    return sc_kernel


def kernel(data, user_factors, movie_factors):
    data_flat = data.astype(jnp.int32).reshape(-1)
    B = data.shape[0]
    V, D = user_factors.shape
    return _make_sc_kernel(B, V, D)(data_flat, user_factors, movie_factors)
